# initial kernel scaffold (unmeasured)
import jax
import jax.numpy as jnp
from jax import lax
from jax.experimental import pallas as pl
from jax.experimental.pallas import tpu as pltpu

N_DEV = 8
H = 8
DH = 128
SCALE = 0.08838834764831843
NEG_INF = -1e30


def kernel(x, Wq, Wo, K_ext, V_ext):
    _, sq, d = x.shape
    _, skv, _, _ = K_ext.shape

    def body(x_ref, wq_ref, wo_ref, k_ref, v_ref, out_ref,
             comm_ref, attn_ref, send_sems, recv_sems):
        my = lax.axis_index("i")
        left = lax.rem(my + N_DEV - 1, N_DEV)
        right = lax.rem(my + 1, N_DEV)

        for hh in range(H):
            comm_ref[0, 0, hh] = k_ref[0, :, hh, :].astype(jnp.bfloat16)
            comm_ref[0, 1, hh] = v_ref[0, :, hh, :].astype(jnp.bfloat16)

        xb = x_ref[0].astype(jnp.bfloat16)
        qs = []
        for hh in range(H):
            wq_h = wq_ref[:, hh * DH:(hh + 1) * DH].astype(jnp.bfloat16)
            q_h = lax.dot_general(xb, wq_h, (((1,), (0,)), ((), ())),
                                  preferred_element_type=jnp.float32)
            qs.append((q_h * SCALE).astype(jnp.bfloat16))

        m = [jnp.full((sq, 1), NEG_INF, jnp.float32) for _ in range(H)]
        l = [jnp.zeros((sq, 1), jnp.float32) for _ in range(H)]
        acc = [jnp.zeros((sq, DH), jnp.float32) for _ in range(H)]

        def process(slot):
            for hh in range(H):
                k_h = comm_ref[slot, 0, hh]
                v_h = comm_ref[slot, 1, hh]
                s = lax.dot_general(qs[hh], k_h, (((1,), (1,)), ((), ())),
                                    preferred_element_type=jnp.float32)
                mj = jnp.max(s, axis=1, keepdims=True)
                m_new = jnp.maximum(m[hh], mj)
                alpha = jnp.exp(m[hh] - m_new)
                p = jnp.exp(s - m_new)
                l[hh] = l[hh] * alpha + jnp.sum(p, axis=1, keepdims=True)
                pv = lax.dot_general(p.astype(jnp.bfloat16), v_h,
                                     (((1,), (0,)), ((), ())),
                                     preferred_element_type=jnp.float32)
                acc[hh] = acc[hh] * alpha + pv
                m[hh] = m_new

        barrier = pltpu.get_barrier_semaphore()
        for nbr in (left, right):
            pl.semaphore_signal(barrier, inc=1, device_id=(nbr,),
                                device_id_type=pl.DeviceIdType.MESH)
        pl.semaphore_wait(barrier, 2)

        for hop in range(N_DEV - 1):
            s_slot = hop % 2
            r_slot = (hop + 1) % 2
            rdma = pltpu.make_async_remote_copy(
                src_ref=comm_ref.at[s_slot],
                dst_ref=comm_ref.at[r_slot],
                send_sem=send_sems.at[s_slot],
                recv_sem=recv_sems.at[r_slot],
                device_id=(right,),
                device_id_type=pl.DeviceIdType.MESH,
            )
            rdma.start()
            process(s_slot)
            rdma.wait()
        process((N_DEV - 1) % 2)

        for hh in range(H):
            attn_ref[:, hh * DH:(hh + 1) * DH] = (
                acc[hh] / l[hh]).astype(jnp.bfloat16)
        out_ref[0] = lax.dot_general(
            attn_ref[...], wo_ref[...].astype(jnp.bfloat16),
            (((1,), (0,)), ((), ())), preferred_element_type=jnp.float32)

    return pl.pallas_call(
        body,
        out_shape=jax.ShapeDtypeStruct((1, sq, d), jnp.float32),
        in_specs=[pl.BlockSpec(memory_space=pltpu.VMEM)] * 5,
        out_specs=pl.BlockSpec(memory_space=pltpu.VMEM),
        scratch_shapes=[
            pltpu.VMEM((2, 2, H, skv, DH), jnp.bfloat16),
            pltpu.VMEM((sq, d), jnp.bfloat16),
            pltpu.SemaphoreType.DMA((2,)),
            pltpu.SemaphoreType.DMA((2,)),
        ],
        compiler_params=pltpu.CompilerParams(collective_id=0),
    )(x, Wq, Wo, K_ext, V_ext)


# baseline (device time: 694534 ns/iter reference)
import jax
import jax.numpy as jnp
from jax import lax
from jax.experimental import pallas as pl
from jax.experimental.pallas import tpu as pltpu

N_DEV = 8
H = 8
DH = 128
SCALE = 0.08838834764831843
NEG_INF = -1e30


def kernel(x, Wq, Wo, K_ext, V_ext):
    _, sq, d = x.shape
    _, skv, _, _ = K_ext.shape

    def body(x_ref, wq_ref, wo_ref, k_ref, v_ref, out_ref,
             comm_ref, stage_ref, attn_ref, q_ref, acc_ref, m_ref, l_ref,
             local_sem, send_sems, recv_sems):
        my = lax.axis_index("i")
        left = lax.rem(my + N_DEV - 1, N_DEV)
        right = lax.rem(my + 1, N_DEV)

        for hh in range(H):
            for kv in range(2):
                src = k_ref if kv == 0 else v_ref
                cp = pltpu.make_async_copy(
                    src.at[0, :, hh, :], stage_ref, local_sem)
                cp.start()
                cp.wait()
                comm_ref[0, kv, hh] = stage_ref[...].astype(jnp.bfloat16)

        xb = x_ref[0].astype(jnp.bfloat16)
        for hh in range(H):
            wq_h = wq_ref[:, hh * DH:(hh + 1) * DH].astype(jnp.bfloat16)
            q_h = lax.dot_general(xb, wq_h, (((1,), (0,)), ((), ())),
                                  preferred_element_type=jnp.float32)
            q_ref[hh] = (q_h * SCALE).astype(jnp.bfloat16)

        m_ref[...] = jnp.full_like(m_ref, NEG_INF)
        l_ref[...] = jnp.zeros_like(l_ref)
        acc_ref[...] = jnp.zeros_like(acc_ref)

        def process(slot):
            for hh in range(H):
                k_h = comm_ref[slot, 0, hh]
                v_h = comm_ref[slot, 1, hh]
                s = lax.dot_general(q_ref[hh], k_h, (((1,), (1,)), ((), ())),
                                    preferred_element_type=jnp.float32)
                mj = jnp.max(s, axis=1, keepdims=True)
                m_old = m_ref[hh]
                m_new = jnp.maximum(m_old, mj)
                alpha = jnp.exp(m_old - m_new)
                p = jnp.exp(s - m_new)
                l_ref[hh] = l_ref[hh] * alpha + jnp.sum(p, axis=1,
                                                        keepdims=True)
                pv = lax.dot_general(p.astype(jnp.bfloat16), v_h,
                                     (((1,), (0,)), ((), ())),
                                     preferred_element_type=jnp.float32)
                acc_ref[hh] = acc_ref[hh] * alpha + pv
                m_ref[hh] = m_new

        barrier = pltpu.get_barrier_semaphore()
        for nbr in (left, right):
            pl.semaphore_signal(barrier, inc=1, device_id=(nbr,),
                                device_id_type=pl.DeviceIdType.MESH)
        pl.semaphore_wait(barrier, 2)

        for hop in range(N_DEV - 1):
            s_slot = hop % 2
            r_slot = (hop + 1) % 2
            rdma = pltpu.make_async_remote_copy(
                src_ref=comm_ref.at[s_slot],
                dst_ref=comm_ref.at[r_slot],
                send_sem=send_sems.at[s_slot],
                recv_sem=recv_sems.at[r_slot],
                device_id=(right,),
                device_id_type=pl.DeviceIdType.MESH,
            )
            rdma.start()
            process(s_slot)
            rdma.wait()
        process((N_DEV - 1) % 2)

        for hh in range(H):
            attn_ref[:, hh * DH:(hh + 1) * DH] = (
                acc_ref[hh] / l_ref[hh]).astype(jnp.bfloat16)
        out_ref[0] = lax.dot_general(
            attn_ref[...], wo_ref[...].astype(jnp.bfloat16),
            (((1,), (0,)), ((), ())), preferred_element_type=jnp.float32)

    return pl.pallas_call(
        body,
        out_shape=jax.ShapeDtypeStruct((1, sq, d), jnp.float32),
        in_specs=[
            pl.BlockSpec(memory_space=pltpu.VMEM),
            pl.BlockSpec(memory_space=pltpu.VMEM),
            pl.BlockSpec(memory_space=pltpu.VMEM),
            pl.BlockSpec(memory_space=pl.ANY),
            pl.BlockSpec(memory_space=pl.ANY),
        ],
        out_specs=pl.BlockSpec(memory_space=pltpu.VMEM),
        scratch_shapes=[
            pltpu.VMEM((2, 2, H, skv, DH), jnp.bfloat16),
            pltpu.VMEM((skv, DH), jnp.float32),
            pltpu.VMEM((sq, d), jnp.bfloat16),
            pltpu.VMEM((H, sq, DH), jnp.bfloat16),
            pltpu.VMEM((H, sq, DH), jnp.float32),
            pltpu.VMEM((H, sq, 1), jnp.float32),
            pltpu.VMEM((H, sq, 1), jnp.float32),
            pltpu.SemaphoreType.DMA,
            pltpu.SemaphoreType.DMA((2,)),
            pltpu.SemaphoreType.DMA((2,)),
        ],
        compiler_params=pltpu.CompilerParams(
            collective_id=0, vmem_limit_bytes=67_000_000),
    )(x, Wq, Wo, K_ext, V_ext)


# device time: 300363 ns/iter; 2.3123x vs baseline; 2.3123x over previous
import jax
import jax.numpy as jnp
from jax import lax
from jax.experimental import pallas as pl
from jax.experimental.pallas import tpu as pltpu

N_DEV = 8
H = 8
DH = 128
SCALE = 0.08838834764831843


def kernel(x, Wq, Wo, K_ext, V_ext):
    _, sq, d = x.shape
    _, skv, _, _ = K_ext.shape

    def body(x_ref, wq_ref, wo_ref, k_ref, v_ref, out_ref,
             kv_ref, stage_ref, qbuf_ref, chbuf_ref, part_ref, own_ref,
             attn_ref, stage_sem, qsend_sems, qrecv_sems, chsend_sems,
             chrecv_sems, credit_sems):
        my = lax.axis_index("i")
        left = lax.rem(my + N_DEV - 1, N_DEV)
        right = lax.rem(my + 1, N_DEV)

        def q_send(j):
            pltpu.make_async_remote_copy(
                src_ref=qbuf_ref.at[j], dst_ref=qbuf_ref.at[j + 1],
                send_sem=qsend_sems.at[j], recv_sem=qrecv_sems.at[j + 1],
                device_id=(right,), device_id_type=pl.DeviceIdType.MESH,
            ).start()

        def q_wait_recv(j):
            pltpu.make_async_remote_copy(
                src_ref=qbuf_ref.at[j], dst_ref=qbuf_ref.at[j],
                send_sem=qrecv_sems.at[j], recv_sem=qrecv_sems.at[j],
                device_id=(left,), device_id_type=pl.DeviceIdType.MESH,
            ).wait_recv()

        def q_wait_send(j):
            pltpu.make_async_remote_copy(
                src_ref=qbuf_ref.at[j], dst_ref=qbuf_ref.at[j],
                send_sem=qsend_sems.at[j], recv_sem=qsend_sems.at[j],
                device_id=(right,), device_id_type=pl.DeviceIdType.MESH,
            ).wait_send()

        def ch_send(j):
            pltpu.make_async_remote_copy(
                src_ref=part_ref.at[j % 2], dst_ref=chbuf_ref.at[(j + 1) % 2],
                send_sem=chsend_sems.at[j % 2],
                recv_sem=chrecv_sems.at[(j + 1) % 2],
                device_id=(right,), device_id_type=pl.DeviceIdType.MESH,
            ).start()

        def ch_wait_send(j):
            pltpu.make_async_remote_copy(
                src_ref=part_ref.at[j % 2], dst_ref=part_ref.at[j % 2],
                send_sem=chsend_sems.at[j % 2], recv_sem=chsend_sems.at[j % 2],
                device_id=(right,), device_id_type=pl.DeviceIdType.MESH,
            ).wait_send()

        def ch_wait_recv(slot):
            pltpu.make_async_remote_copy(
                src_ref=chbuf_ref.at[slot], dst_ref=chbuf_ref.at[slot],
                send_sem=chrecv_sems.at[slot], recv_sem=chrecv_sems.at[slot],
                device_id=(left,), device_id_type=pl.DeviceIdType.MESH,
            ).wait_recv()

        for hh in range(H):
            for kv in range(2):
                src = k_ref if kv == 0 else v_ref
                cp = pltpu.make_async_copy(
                    src.at[0, :, hh, :], stage_ref, stage_sem)
                cp.start()
                cp.wait()
                kv_ref[kv, hh] = stage_ref[...].astype(jnp.bfloat16)

        xb = x_ref[0].astype(jnp.bfloat16)
        for hh in range(H):
            wq_h = wq_ref[:, hh * DH:(hh + 1) * DH].astype(jnp.bfloat16)
            q_h = lax.dot_general(xb, wq_h, (((1,), (0,)), ((), ())),
                                  preferred_element_type=jnp.float32)
            qbuf_ref[0, hh] = (q_h * SCALE).astype(jnp.bfloat16)

        def compute_partial(qslot, dst_ref):
            for hh in range(H):
                q_h = qbuf_ref[qslot, hh]
                s_mat = lax.dot_general(
                    q_h, kv_ref[0, hh], (((1,), (1,)), ((), ())),
                    preferred_element_type=jnp.float32)
                p = jnp.exp(s_mat)
                dst_ref[hh] = lax.dot_general(
                    p.astype(jnp.bfloat16), kv_ref[1, hh],
                    (((1,), (0,)), ((), ())),
                    preferred_element_type=jnp.float32)
                dst_ref[H, :, hh:hh + 1] = jnp.sum(p, axis=1, keepdims=True)

        barrier = pltpu.get_barrier_semaphore()
        for nbr in (left, right):
            pl.semaphore_signal(barrier, inc=1, device_id=(nbr,),
                                device_id_type=pl.DeviceIdType.MESH)
        pl.semaphore_wait(barrier, 2)

        q_send(0)
        compute_partial(0, own_ref)

        for j in range(1, N_DEV):
            s = j % 2
            q_wait_recv(j)
            if j <= N_DEV - 2:
                q_send(j)
            if j >= 3:
                ch_wait_send(j - 2)
            compute_partial(j, part_ref.at[s])
            if j >= 2:
                ch_wait_recv(s)
                part_ref[s] = part_ref[s] + chbuf_ref[s]
                if j <= N_DEV - 2:
                    pl.semaphore_signal(
                        credit_sems.at[s], inc=1, device_id=(left,),
                        device_id_type=pl.DeviceIdType.MESH)
            if j >= 3:
                pl.semaphore_wait(credit_sems.at[(j + 1) % 2], 1)
            ch_send(j)

        ch_wait_recv(0)
        for j in range(N_DEV - 1):
            q_wait_send(j)
        ch_wait_send(N_DEV - 2)
        ch_wait_send(N_DEV - 1)

        tot = chbuf_ref[0] + own_ref[...]
        for hh in range(H):
            attn_ref[:, hh * DH:(hh + 1) * DH] = (
                tot[hh] / tot[H, :, hh:hh + 1]).astype(jnp.bfloat16)
        out_ref[0] = lax.dot_general(
            attn_ref[...], wo_ref[...].astype(jnp.bfloat16),
            (((1,), (0,)), ((), ())), preferred_element_type=jnp.float32)

    return pl.pallas_call(
        body,
        out_shape=jax.ShapeDtypeStruct((1, sq, d), jnp.float32),
        in_specs=[
            pl.BlockSpec(memory_space=pltpu.VMEM),
            pl.BlockSpec(memory_space=pltpu.VMEM),
            pl.BlockSpec(memory_space=pltpu.VMEM),
            pl.BlockSpec(memory_space=pl.ANY),
            pl.BlockSpec(memory_space=pl.ANY),
        ],
        out_specs=pl.BlockSpec(memory_space=pltpu.VMEM),
        scratch_shapes=[
            pltpu.VMEM((2, H, skv, DH), jnp.bfloat16),
            pltpu.VMEM((skv, DH), jnp.float32),
            pltpu.VMEM((N_DEV, H, sq, DH), jnp.bfloat16),
            pltpu.VMEM((2, H + 1, sq, DH), jnp.float32),
            pltpu.VMEM((2, H + 1, sq, DH), jnp.float32),
            pltpu.VMEM((H + 1, sq, DH), jnp.float32),
            pltpu.VMEM((sq, d), jnp.bfloat16),
            pltpu.SemaphoreType.DMA,
            pltpu.SemaphoreType.DMA((N_DEV,)),
            pltpu.SemaphoreType.DMA((N_DEV,)),
            pltpu.SemaphoreType.DMA((2,)),
            pltpu.SemaphoreType.DMA((2,)),
            pltpu.SemaphoreType.REGULAR((2,)),
        ],
        compiler_params=pltpu.CompilerParams(
            collective_id=0, vmem_limit_bytes=67_000_000),
    )(x, Wq, Wo, K_ext, V_ext)


# device time: 198653 ns/iter; 3.4962x vs baseline; 1.5120x over previous
import jax
import jax.numpy as jnp
from jax import lax
from jax.experimental import pallas as pl
from jax.experimental.pallas import tpu as pltpu

N_DEV = 8
H = 8
DH = 128
SCALE = 0.08838834764831843


def kernel(x, Wq, Wo, K_ext, V_ext):
    _, sq, d = x.shape
    _, skv, _, _ = K_ext.shape

    def body(x_ref, wq_ref, wo_ref, k_ref, v_ref, out_ref,
             kv_ref, stage_ref, qbuf_ref, chbuf_ref, sendbuf_ref, part_ref,
             own_ref, attn_ref, stage_sems, qsend_sems, qrecv_sems,
             chsend_sems, chrecv_sems, credit_sems):
        my = lax.axis_index("i")
        left = lax.rem(my + N_DEV - 1, N_DEV)
        right = lax.rem(my + 1, N_DEV)

        def q_send(j):
            pltpu.make_async_remote_copy(
                src_ref=qbuf_ref.at[j], dst_ref=qbuf_ref.at[j + 1],
                send_sem=qsend_sems.at[j], recv_sem=qrecv_sems.at[j + 1],
                device_id=(right,), device_id_type=pl.DeviceIdType.MESH,
            ).start()

        def q_wait_recv(j):
            pltpu.make_async_remote_copy(
                src_ref=qbuf_ref.at[j], dst_ref=qbuf_ref.at[j],
                send_sem=qrecv_sems.at[j], recv_sem=qrecv_sems.at[j],
                device_id=(left,), device_id_type=pl.DeviceIdType.MESH,
            ).wait_recv()

        def q_wait_send(j):
            pltpu.make_async_remote_copy(
                src_ref=qbuf_ref.at[j], dst_ref=qbuf_ref.at[j],
                send_sem=qsend_sems.at[j], recv_sem=qsend_sems.at[j],
                device_id=(right,), device_id_type=pl.DeviceIdType.MESH,
            ).wait_send()

        def ch_send(j):
            pltpu.make_async_remote_copy(
                src_ref=sendbuf_ref.at[j % 2],
                dst_ref=chbuf_ref.at[(j + 1) % 2],
                send_sem=chsend_sems.at[j % 2],
                recv_sem=chrecv_sems.at[(j + 1) % 2],
                device_id=(right,), device_id_type=pl.DeviceIdType.MESH,
            ).start()

        def ch_wait_send(j):
            pltpu.make_async_remote_copy(
                src_ref=sendbuf_ref.at[j % 2], dst_ref=sendbuf_ref.at[j % 2],
                send_sem=chsend_sems.at[j % 2], recv_sem=chsend_sems.at[j % 2],
                device_id=(right,), device_id_type=pl.DeviceIdType.MESH,
            ).wait_send()

        def ch_wait_recv(slot):
            pltpu.make_async_remote_copy(
                src_ref=chbuf_ref.at[slot], dst_ref=chbuf_ref.at[slot],
                send_sem=chrecv_sems.at[slot], recv_sem=chrecv_sems.at[slot],
                device_id=(left,), device_id_type=pl.DeviceIdType.MESH,
            ).wait_recv()

        xb = x_ref[0].astype(jnp.bfloat16)
        for hh in range(H):
            wq_h = wq_ref[:, hh * DH:(hh + 1) * DH].astype(jnp.bfloat16)
            q_h = lax.dot_general(xb, wq_h, (((1,), (0,)), ((), ())),
                                  preferred_element_type=jnp.float32)
            qbuf_ref[0, hh] = (q_h * SCALE).astype(jnp.bfloat16)

        def compute_partial(qslot, dst_ref):
            for hh in range(H):
                q_h = qbuf_ref[qslot, hh]
                s_mat = lax.dot_general(
                    q_h, kv_ref[0, hh], (((1,), (1,)), ((), ())),
                    preferred_element_type=jnp.float32)
                p = jnp.exp(s_mat)
                dst_ref[hh] = lax.dot_general(
                    p.astype(jnp.bfloat16), kv_ref[1, hh],
                    (((1,), (0,)), ((), ())),
                    preferred_element_type=jnp.float32)
                dst_ref[H, :, hh:hh + 1] = jnp.sum(p, axis=1, keepdims=True)

        barrier = pltpu.get_barrier_semaphore()
        for nbr in (left, right):
            pl.semaphore_signal(barrier, inc=1, device_id=(nbr,),
                                device_id_type=pl.DeviceIdType.MESH)
        pl.semaphore_wait(barrier, 2)

        q_send(0)

        def stage_start(idx):
            hh, kv = divmod(idx, 2)
            src = k_ref if kv == 0 else v_ref
            cp = pltpu.make_async_copy(
                src.at[0, :, hh, :], stage_ref.at[idx % 2],
                stage_sems.at[idx % 2])
            cp.start()
            return cp

        def stage_finish(idx, cp):
            hh, kv = divmod(idx, 2)
            cp.wait()
            kv_ref[kv, hh] = stage_ref[idx % 2].astype(jnp.bfloat16)

        copies = {0: stage_start(0), 1: stage_start(1)}
        for idx in range(2, 2 * H):
            stage_finish(idx - 2, copies[idx - 2])
            copies[idx] = stage_start(idx)
        stage_finish(2 * H - 2, copies[2 * H - 2])
        stage_finish(2 * H - 1, copies[2 * H - 1])

        compute_partial(0, own_ref)

        for j in range(1, N_DEV):
            s = j % 2
            q_wait_recv(j)
            if j <= N_DEV - 2:
                q_send(j)
            compute_partial(j, part_ref.at[s])
            if j >= 2:
                ch_wait_recv(s)
                merged = part_ref[s] + chbuf_ref[s].astype(jnp.float32)
                if j <= N_DEV - 2:
                    pl.semaphore_signal(
                        credit_sems.at[s], inc=1, device_id=(left,),
                        device_id_type=pl.DeviceIdType.MESH)
            else:
                merged = part_ref[s]
            if j >= 3:
                ch_wait_send(j - 2)
            sendbuf_ref[s] = merged.astype(jnp.bfloat16)
            if j >= 3:
                pl.semaphore_wait(credit_sems.at[(j + 1) % 2], 1)
            ch_send(j)

        ch_wait_recv(0)
        for j in range(N_DEV - 1):
            q_wait_send(j)
        ch_wait_send(N_DEV - 2)
        ch_wait_send(N_DEV - 1)

        tot = chbuf_ref[0].astype(jnp.float32) + own_ref[...]
        for hh in range(H):
            attn_ref[:, hh * DH:(hh + 1) * DH] = (
                tot[hh] / tot[H, :, hh:hh + 1]).astype(jnp.bfloat16)
        out_ref[0] = lax.dot_general(
            attn_ref[...], wo_ref[...].astype(jnp.bfloat16),
            (((1,), (0,)), ((), ())), preferred_element_type=jnp.float32)

    return pl.pallas_call(
        body,
        out_shape=jax.ShapeDtypeStruct((1, sq, d), jnp.float32),
        in_specs=[
            pl.BlockSpec(memory_space=pltpu.VMEM),
            pl.BlockSpec(memory_space=pltpu.VMEM),
            pl.BlockSpec(memory_space=pltpu.VMEM),
            pl.BlockSpec(memory_space=pl.ANY),
            pl.BlockSpec(memory_space=pl.ANY),
        ],
        out_specs=pl.BlockSpec(memory_space=pltpu.VMEM),
        scratch_shapes=[
            pltpu.VMEM((2, H, skv, DH), jnp.bfloat16),
            pltpu.VMEM((2, skv, DH), jnp.float32),
            pltpu.VMEM((N_DEV, H, sq, DH), jnp.bfloat16),
            pltpu.VMEM((2, H + 1, sq, DH), jnp.bfloat16),
            pltpu.VMEM((2, H + 1, sq, DH), jnp.bfloat16),
            pltpu.VMEM((2, H + 1, sq, DH), jnp.float32),
            pltpu.VMEM((H + 1, sq, DH), jnp.float32),
            pltpu.VMEM((sq, d), jnp.bfloat16),
            pltpu.SemaphoreType.DMA((2,)),
            pltpu.SemaphoreType.DMA((N_DEV,)),
            pltpu.SemaphoreType.DMA((N_DEV,)),
            pltpu.SemaphoreType.DMA((2,)),
            pltpu.SemaphoreType.DMA((2,)),
            pltpu.SemaphoreType.REGULAR((2,)),
        ],
        compiler_params=pltpu.CompilerParams(
            collective_id=0, vmem_limit_bytes=67_000_000),
    )(x, Wq, Wo, K_ext, V_ext)


# device time: 171474 ns/iter; 4.0504x vs baseline; 1.1585x over previous
import jax
import jax.numpy as jnp
from jax import lax
from jax.experimental import pallas as pl
from jax.experimental.pallas import tpu as pltpu

N_DEV = 8
H = 8
DH = 128
SCALE = 0.08838834764831843


def kernel(x, Wq, Wo, K_ext, V_ext):
    _, sq, d = x.shape
    _, skv, _, _ = K_ext.shape

    def body(x_ref, wq_ref, wo_ref, k_ref, v_ref, out_ref,
             kv_ref, stage_ref, qbuf_ref, chbuf_ref, sendbuf_ref, part_ref,
             own_ref, attn_ref, stage_sems, qsend_sems, qrecv_sems,
             chsend_sems, chrecv_sems, credit_sems):
        my = lax.axis_index("i")
        left = lax.rem(my + N_DEV - 1, N_DEV)
        right = lax.rem(my + 1, N_DEV)

        def q_send(src_slot, dst_slot, sem_idx, dev):
            pltpu.make_async_remote_copy(
                src_ref=qbuf_ref.at[src_slot], dst_ref=qbuf_ref.at[dst_slot],
                send_sem=qsend_sems.at[sem_idx],
                recv_sem=qrecv_sems.at[dst_slot],
                device_id=(dev,), device_id_type=pl.DeviceIdType.MESH,
            ).start()

        def q_wait_recv(slot):
            pltpu.make_async_remote_copy(
                src_ref=qbuf_ref.at[slot], dst_ref=qbuf_ref.at[slot],
                send_sem=qrecv_sems.at[slot], recv_sem=qrecv_sems.at[slot],
                device_id=(left,), device_id_type=pl.DeviceIdType.MESH,
            ).wait_recv()

        def q_wait_send(sem_idx):
            pltpu.make_async_remote_copy(
                src_ref=qbuf_ref.at[0], dst_ref=qbuf_ref.at[0],
                send_sem=qsend_sems.at[sem_idx], recv_sem=qsend_sems.at[sem_idx],
                device_id=(right,), device_id_type=pl.DeviceIdType.MESH,
            ).wait_send()

        def ch_send(j):
            pltpu.make_async_remote_copy(
                src_ref=sendbuf_ref.at[j % 2],
                dst_ref=chbuf_ref.at[(j + 1) % 2],
                send_sem=chsend_sems.at[j % 2],
                recv_sem=chrecv_sems.at[(j + 1) % 2],
                device_id=(right,), device_id_type=pl.DeviceIdType.MESH,
            ).start()

        def ch_wait_send(j):
            pltpu.make_async_remote_copy(
                src_ref=sendbuf_ref.at[j % 2], dst_ref=sendbuf_ref.at[j % 2],
                send_sem=chsend_sems.at[j % 2], recv_sem=chsend_sems.at[j % 2],
                device_id=(right,), device_id_type=pl.DeviceIdType.MESH,
            ).wait_send()

        def ch_wait_recv(slot):
            pltpu.make_async_remote_copy(
                src_ref=chbuf_ref.at[slot], dst_ref=chbuf_ref.at[slot],
                send_sem=chrecv_sems.at[slot], recv_sem=chrecv_sems.at[slot],
                device_id=(left,), device_id_type=pl.DeviceIdType.MESH,
            ).wait_recv()

        xb = x_ref[0].astype(jnp.bfloat16)
        for hh in range(H):
            wq_h = wq_ref[:, hh * DH:(hh + 1) * DH].astype(jnp.bfloat16)
            q_h = lax.dot_general(xb, wq_h, (((1,), (0,)), ((), ())),
                                  preferred_element_type=jnp.float32)
            qbuf_ref[0, hh] = (q_h * SCALE).astype(jnp.bfloat16)

        def compute_partial(qslot, dst_ref):
            for hh in range(H):
                q_h = qbuf_ref[qslot, hh]
                s_mat = lax.dot_general(
                    q_h, kv_ref[0, hh], (((1,), (1,)), ((), ())),
                    preferred_element_type=jnp.float32)
                p = jnp.exp(s_mat)
                dst_ref[hh] = lax.dot_general(
                    p.astype(jnp.bfloat16), kv_ref[1, hh],
                    (((1,), (0,)), ((), ())),
                    preferred_element_type=jnp.float32)
                dst_ref[H, :, hh:hh + 1] = jnp.sum(p, axis=1, keepdims=True)

        barrier = pltpu.get_barrier_semaphore()
        for nbr in (left, right):
            pl.semaphore_signal(barrier, inc=1, device_id=(nbr,),
                                device_id_type=pl.DeviceIdType.MESH)
        pl.semaphore_wait(barrier, 2)

        q_send(0, 1, 0, right)
        q_send(0, N_DEV - 1, 5, left)

        def stage_start(idx):
            hh, kv = divmod(idx, 2)
            src = k_ref if kv == 0 else v_ref
            cp = pltpu.make_async_copy(
                src.at[0, :, hh, :], stage_ref.at[idx % 2],
                stage_sems.at[idx % 2])
            cp.start()
            return cp

        def stage_finish(idx, cp):
            hh, kv = divmod(idx, 2)
            cp.wait()
            kv_ref[kv, hh] = stage_ref[idx % 2].astype(jnp.bfloat16)

        copies = {0: stage_start(0), 1: stage_start(1)}
        for idx in range(2, 2 * H):
            stage_finish(idx - 2, copies[idx - 2])
            copies[idx] = stage_start(idx)
        stage_finish(2 * H - 2, copies[2 * H - 2])
        stage_finish(2 * H - 1, copies[2 * H - 1])

        compute_partial(0, own_ref)

        for j in range(1, N_DEV):
            s = j % 2
            if j <= 5:
                q_wait_recv(j)
            if j <= 3:
                q_send(j, j + 1, j, right)
            if j == 1:
                q_wait_recv(7)
                q_send(7, 6, 6, left)
            if j == 2:
                q_wait_recv(6)
                q_send(6, 5, 7, left)
            compute_partial(j, part_ref.at[s])
            if j >= 2:
                ch_wait_recv(s)
                merged = part_ref[s] + chbuf_ref[s].astype(jnp.float32)
                if j <= N_DEV - 2:
                    pl.semaphore_signal(
                        credit_sems.at[s], inc=1, device_id=(left,),
                        device_id_type=pl.DeviceIdType.MESH)
            else:
                merged = part_ref[s]
            if j >= 3:
                ch_wait_send(j - 2)
            sendbuf_ref[s] = merged.astype(jnp.bfloat16)
            if j >= 3:
                pl.semaphore_wait(credit_sems.at[(j + 1) % 2], 1)
            ch_send(j)

        ch_wait_recv(0)
        for sem_idx in (0, 1, 2, 3, 5, 6, 7):
            q_wait_send(sem_idx)
        ch_wait_send(N_DEV - 2)
        ch_wait_send(N_DEV - 1)

        tot = chbuf_ref[0].astype(jnp.float32) + own_ref[...]
        for hh in range(H):
            attn_ref[:, hh * DH:(hh + 1) * DH] = (
                tot[hh] / tot[H, :, hh:hh + 1]).astype(jnp.bfloat16)
        out_ref[0] = lax.dot_general(
            attn_ref[...], wo_ref[...].astype(jnp.bfloat16),
            (((1,), (0,)), ((), ())), preferred_element_type=jnp.float32)

    return pl.pallas_call(
        body,
        out_shape=jax.ShapeDtypeStruct((1, sq, d), jnp.float32),
        in_specs=[
            pl.BlockSpec(memory_space=pltpu.VMEM),
            pl.BlockSpec(memory_space=pltpu.VMEM),
            pl.BlockSpec(memory_space=pltpu.VMEM),
            pl.BlockSpec(memory_space=pl.ANY),
            pl.BlockSpec(memory_space=pl.ANY),
        ],
        out_specs=pl.BlockSpec(memory_space=pltpu.VMEM),
        scratch_shapes=[
            pltpu.VMEM((2, H, skv, DH), jnp.bfloat16),
            pltpu.VMEM((2, skv, DH), jnp.float32),
            pltpu.VMEM((N_DEV, H, sq, DH), jnp.bfloat16),
            pltpu.VMEM((2, H + 1, sq, DH), jnp.bfloat16),
            pltpu.VMEM((2, H + 1, sq, DH), jnp.bfloat16),
            pltpu.VMEM((2, H + 1, sq, DH), jnp.float32),
            pltpu.VMEM((H + 1, sq, DH), jnp.float32),
            pltpu.VMEM((sq, d), jnp.bfloat16),
            pltpu.SemaphoreType.DMA((2,)),
            pltpu.SemaphoreType.DMA((N_DEV,)),
            pltpu.SemaphoreType.DMA((N_DEV,)),
            pltpu.SemaphoreType.DMA((2,)),
            pltpu.SemaphoreType.DMA((2,)),
            pltpu.SemaphoreType.REGULAR((2,)),
        ],
        compiler_params=pltpu.CompilerParams(
            collective_id=0, vmem_limit_bytes=67_000_000),
    )(x, Wq, Wo, K_ext, V_ext)
